# Initial kernel scaffold; baseline (speedup 1.0000x reference)
import jax
import jax.numpy as jnp
from jax.experimental import pallas as pl


def kernel(x1, x2, edge_index1, edge_index2, W):
    raise NotImplementedError("write your pallas kernel here")


# speedup vs baseline: 1.0062x; 1.0062x over previous
import jax
import jax.numpy as jnp
from jax.experimental import pallas as pl

N = 10000
D = 128
H = 128


def _mm_kernel(x_ref, w_ref, o_ref):
    o_ref[...] = jnp.dot(x_ref[...], w_ref[...], preferred_element_type=jnp.float32)


def _matmul(x, W):
    return pl.pallas_call(
        _mm_kernel,
        out_shape=jax.ShapeDtypeStruct((N, H), jnp.float32),
        grid=(10,),
        in_specs=[
            pl.BlockSpec((1000, D), lambda i: (i, 0)),
            pl.BlockSpec((D, H), lambda i: (0, 0)),
        ],
        out_specs=pl.BlockSpec((1000, H), lambda i: (i, 0)),
    )(x, W)


def _gcn(x, src, dst, W):
    ones = jnp.ones((src.shape[0],), dtype=x.dtype)
    deg_out = jnp.zeros((N,), dtype=x.dtype).at[src].add(ones)
    deg_in = jnp.zeros((N,), dtype=x.dtype).at[dst].add(ones)
    norm = jax.lax.rsqrt(jnp.maximum(deg_out, 1.0)[src] * jnp.maximum(deg_in, 1.0)[dst])
    h = _matmul(x, W)
    msgs = jnp.take(h, src, axis=0) * norm[:, None]
    agg = jnp.zeros((N, H), dtype=x.dtype).at[dst].add(msgs)
    return jax.nn.relu(agg)


def kernel(x1, x2, edge_index1, edge_index2, W):
    h1 = _gcn(x1, edge_index1[0], edge_index1[1], W)
    h2 = _gcn(x2, edge_index2[0], edge_index2[1], W)
    e1 = jnp.mean(h1, axis=0)
    e2 = jnp.mean(h2, axis=0)
    return jnp.sum(e1 * e2)


# speedup vs baseline: 15.3597x; 15.2651x over previous
import functools

import jax
import jax.numpy as jnp
from jax import lax
from jax.experimental import pallas as pl
from jax.experimental.pallas import tpu as pltpu
from jax.experimental.pallas import tpu_sc as plsc

N = 10000
D = 128
H = 128
E = 320000

N_PAD = 10240
CHUNK = 128
NT = 16
T_TILE = 160
T_PAD = T_TILE * NT
E_PAD = T_PAD * CHUNK
ROWS_T = N_PAD // NT
RB = 64
NRB = ROWS_T // RB
SUP = 16


def _hist_body(idx_hbm, deg_hbm, idx_v, ones_v, zeros_v, degs_sh, degd_sh, sem):
    c = lax.axis_index("c")
    s = lax.axis_index("s")
    for i in range(CHUNK // 16):
        ones_v[pl.ds(i * 16, 16)] = jnp.ones((16,), jnp.float32)
    for i in range(ROWS_T // 16):
        zeros_v[pl.ds(i * 16, 16)] = jnp.zeros((16,), jnp.float32)
    pltpu.sync_copy(zeros_v, degs_sh.at[pl.ds(s * ROWS_T, ROWS_T)])
    pltpu.sync_copy(zeros_v, degd_sh.at[pl.ds(s * ROWS_T, ROWS_T)])
    plsc.subcore_barrier()

    pltpu.sync_copy(idx_hbm.at[2 * c, pl.ds(s * T_TILE, T_TILE), :], idx_v.at[0])
    pltpu.sync_copy(idx_hbm.at[2 * c + 1, pl.ds(s * T_TILE, T_TILE), :], idx_v.at[1])

    def body(j, carry):
        pltpu.sync_copy(ones_v, degs_sh.at[idx_v.at[0, j]], add=True)
        pltpu.sync_copy(ones_v, degd_sh.at[idx_v.at[1, j]], add=True)
        return carry

    lax.fori_loop(0, T_TILE, body, 0)
    plsc.subcore_barrier()

    @pl.when(s == 0)
    def _():
        pltpu.sync_copy(degs_sh, deg_hbm.at[2 * c, 0])
        pltpu.sync_copy(degd_sh, deg_hbm.at[2 * c + 1, 0])


def _degree_hist(idx4):
    mesh = plsc.VectorSubcoreMesh(core_axis_name="c", subcore_axis_name="s")
    f = functools.partial(
        pl.kernel,
        out_type=jax.ShapeDtypeStruct((4, 1, N_PAD), jnp.float32),
        mesh=mesh,
        scratch_types=[
            pltpu.VMEM((2, T_TILE, CHUNK), jnp.int32),
            pltpu.VMEM((CHUNK,), jnp.float32),
            pltpu.VMEM((ROWS_T,), jnp.float32),
            pltpu.VMEM_SHARED((N_PAD,), jnp.float32),
            pltpu.VMEM_SHARED((N_PAD,), jnp.float32),
            pltpu.SemaphoreType.DMA,
        ],
    )(_hist_body)
    return f(idx4)


def _mm_body(x_ref, w_ref, dega_ref, degb_ref, h_ref, b_ref):
    a = lax.rsqrt(jnp.maximum(dega_ref[0], 1.0))
    h = jnp.dot(x_ref[...], w_ref[...], preferred_element_type=jnp.float32)
    h_ref[...] = h * a
    b_ref[...] = lax.rsqrt(jnp.maximum(degb_ref[...], 1.0))


def _matmul_scale(x_flat, W, dega3, degb2):
    nblk = (2 * N_PAD) // 256
    return pl.pallas_call(
        _mm_body,
        out_shape=(
            jax.ShapeDtypeStruct((2 * N_PAD, H), jnp.float32),
            jax.ShapeDtypeStruct((nblk, 1, 256), jnp.float32),
        ),
        grid=(nblk,),
        in_specs=[
            pl.BlockSpec((256, D), lambda i: (i, 0)),
            pl.BlockSpec((D, H), lambda i: (0, 0)),
            pl.BlockSpec((1, 256, 1), lambda i: (i, 0, 0)),
            pl.BlockSpec((1, 1, 256), lambda i: (i, 0, 0)),
        ],
        out_specs=(
            pl.BlockSpec((256, H), lambda i: (i, 0)),
            pl.BlockSpec((1, 1, 256), lambda i: (i, 0, 0)),
        ),
    )(x_flat, W, dega3, degb2)


def _agg_body(h_hbm, srcsh_hbm, dst_hbm, b_hbm, e_hbm,
              idxs_v, idxd_v, rows_v, red_v, b_v, acc_v, pbuf_v,
              agg_sh, part_sh, sem):
    c = lax.axis_index("c")
    s = lax.axis_index("s")

    for r in range(RB):
        for k in range(H // 16):
            red_v[r, pl.ds(k * 16, 16)] = jnp.zeros((16,), jnp.float32)
    for i in range(NRB):
        pltpu.sync_copy(red_v, agg_sh.at[pl.ds(s * ROWS_T + i * RB, RB), :])
    pltpu.sync_copy(red_v.at[0], part_sh.at[s, 0])
    plsc.subcore_barrier()

    def super_body(u, carry):
        base = s * T_TILE + u * SUP
        pltpu.sync_copy(srcsh_hbm.at[c, pl.ds(base, SUP), :], idxs_v)
        pltpu.sync_copy(dst_hbm.at[c, pl.ds(base, SUP), :], idxd_v)

        def body(j, carry2):
            pltpu.async_copy(h_hbm.at[idxs_v.at[j]], rows_v, sem).wait()
            pltpu.sync_copy(rows_v, agg_sh.at[idxd_v.at[j]], add=True)
            return carry2

        lax.fori_loop(0, SUP, body, 0)
        return carry

    lax.fori_loop(0, T_TILE // SUP, super_body, 0)
    plsc.subcore_barrier()

    pltpu.sync_copy(b_hbm.at[c, 0, pl.ds(s * ROWS_T, ROWS_T)], b_v)

    def red_block(i, acc):
        pltpu.sync_copy(agg_sh.at[pl.ds(s * ROWS_T + i * RB, RB), :], red_v)
        for g in range(RB // 16):
            bv = b_v[pl.ds(i * RB + g * 16, 16)]
            for r in range(16):
                bs = bv[r]
                for k in range(H // 16):
                    v = red_v[g * 16 + r, pl.ds(k * 16, 16)]
                    acc = tuple(
                        acc[q] + jnp.maximum(v, 0.0) * bs if q == k else acc[q]
                        for q in range(H // 16)
                    )
        return acc

    acc0 = tuple(jnp.zeros((16,), jnp.float32) for _ in range(H // 16))
    acc = lax.fori_loop(0, NRB, red_block, acc0)
    for k in range(H // 16):
        acc_v[pl.ds(k * 16, 16)] = acc[k]
    pltpu.sync_copy(acc_v, part_sh.at[s, 0])
    plsc.subcore_barrier()

    @pl.when(s == 0)
    def _():
        pltpu.sync_copy(part_sh, pbuf_v)
        tot = tuple(jnp.zeros((16,), jnp.float32) for _ in range(H // 16))
        for t in range(NT):
            for k in range(H // 16):
                tot = tuple(
                    tot[q] + pbuf_v[t, 0, pl.ds(k * 16, 16)] if q == k else tot[q]
                    for q in range(H // 16)
                )
        for k in range(H // 16):
            acc_v[pl.ds(k * 16, 16)] = tot[k]
        pltpu.sync_copy(acc_v, e_hbm.at[c, 0])


def _aggregate(h_flat, srcsh, dst2, b3):
    mesh = plsc.VectorSubcoreMesh(core_axis_name="c", subcore_axis_name="s")
    f = functools.partial(
        pl.kernel,
        out_type=jax.ShapeDtypeStruct((2, 1, H), jnp.float32),
        mesh=mesh,
        scratch_types=[
            pltpu.VMEM((SUP, CHUNK), jnp.int32),
            pltpu.VMEM((SUP, CHUNK), jnp.int32),
            pltpu.VMEM((CHUNK, H), jnp.float32),
            pltpu.VMEM((RB, H), jnp.float32),
            pltpu.VMEM((ROWS_T,), jnp.float32),
            pltpu.VMEM((H,), jnp.float32),
            pltpu.VMEM((NT, 1, H), jnp.float32),
            pltpu.VMEM_SHARED((N_PAD, H), jnp.float32),
            pltpu.VMEM_SHARED((NT, 1, H), jnp.float32),
            pltpu.SemaphoreType.DMA,
        ],
    )(_agg_body)
    return f(h_flat, srcsh, dst2, b3)


def _dot_body(e_ref, o_ref):
    o_ref[...] = jnp.sum(e_ref[0] * e_ref[1], keepdims=True)[None] * (
        1.0 / (float(N) * float(N))
    )


def _dot(e2):
    return pl.pallas_call(
        _dot_body,
        out_shape=jax.ShapeDtypeStruct((1, 1), jnp.float32),
    )(e2)


def _pad_idx(a):
    return jnp.concatenate([a, jnp.full((E_PAD - E,), N, jnp.int32)])


def kernel(x1, x2, edge_index1, edge_index2, W):
    ei1 = edge_index1.astype(jnp.int32)
    ei2 = edge_index2.astype(jnp.int32)
    src1, dst1 = _pad_idx(ei1[0]), _pad_idx(ei1[1])
    src2, dst2 = _pad_idx(ei2[0]), _pad_idx(ei2[1])
    idx4 = jnp.stack([src1, dst1, src2, dst2]).reshape(4, T_PAD, CHUNK)
    srcsh = jnp.stack([src1, src2 + N_PAD]).reshape(2, T_PAD, CHUNK)
    dsts = jnp.stack([dst1, dst2]).reshape(2, T_PAD, CHUNK)

    xp1 = jnp.pad(x1, ((0, N_PAD - N), (0, 0)))
    xp2 = jnp.pad(x2, ((0, N_PAD - N), (0, 0)))
    x_flat = jnp.concatenate([xp1, xp2])

    deg = _degree_hist(idx4).reshape(4, N_PAD)
    dega3 = deg[jnp.array([0, 2])].reshape((2 * N_PAD) // 256, 256, 1)
    degb2 = deg[jnp.array([1, 3])].reshape((2 * N_PAD) // 256, 1, 256)

    h_flat, b2 = _matmul_scale(x_flat, W, dega3, degb2)
    b3 = b2.reshape(2, 1, N_PAD)

    e2 = _aggregate(h_flat, srcsh, dsts, b3)
    return _dot(e2.reshape(2, H))[0, 0]


# speedup vs baseline: 17.8203x; 1.1602x over previous
import functools

import jax
import jax.numpy as jnp
from jax import lax
from jax.experimental import pallas as pl
from jax.experimental.pallas import tpu as pltpu
from jax.experimental.pallas import tpu_sc as plsc

N = 10000
D = 128
H = 128
E = 320000

N_PAD = 10240
CHUNK = 128
NT = 16
T_TILE = 160
T_PAD = T_TILE * NT
E_PAD = T_PAD * CHUNK
ROWS_T = N_PAD // NT
RB = 64
NRB = ROWS_T // RB
SUP = 16


def _hist_body(idx_hbm, deg_hbm, idx_v, ones_v, zeros_v, degs_sh, degd_sh, sem):
    c = lax.axis_index("c")
    s = lax.axis_index("s")
    for i in range(CHUNK // 16):
        ones_v[pl.ds(i * 16, 16)] = jnp.ones((16,), jnp.float32)
    for i in range(ROWS_T // 16):
        zeros_v[pl.ds(i * 16, 16)] = jnp.zeros((16,), jnp.float32)
    pltpu.sync_copy(zeros_v, degs_sh.at[pl.ds(s * ROWS_T, ROWS_T)])
    pltpu.sync_copy(zeros_v, degd_sh.at[pl.ds(s * ROWS_T, ROWS_T)])
    plsc.subcore_barrier()

    pltpu.sync_copy(idx_hbm.at[2 * c, pl.ds(s * T_TILE, T_TILE), :], idx_v.at[0])
    pltpu.sync_copy(idx_hbm.at[2 * c + 1, pl.ds(s * T_TILE, T_TILE), :], idx_v.at[1])

    def body(j, carry):
        pltpu.sync_copy(ones_v, degs_sh.at[idx_v.at[0, j]], add=True)
        pltpu.sync_copy(ones_v, degd_sh.at[idx_v.at[1, j]], add=True)
        return carry

    lax.fori_loop(0, T_TILE, body, 0)
    plsc.subcore_barrier()

    @pl.when(s == 0)
    def _():
        pltpu.sync_copy(degs_sh, deg_hbm.at[2 * c, 0])
        pltpu.sync_copy(degd_sh, deg_hbm.at[2 * c + 1, 0])


def _degree_hist(idx4):
    mesh = plsc.VectorSubcoreMesh(core_axis_name="c", subcore_axis_name="s")
    f = functools.partial(
        pl.kernel,
        out_type=jax.ShapeDtypeStruct((4, 1, N_PAD), jnp.float32),
        mesh=mesh,
        scratch_types=[
            pltpu.VMEM((2, T_TILE, CHUNK), jnp.int32),
            pltpu.VMEM((CHUNK,), jnp.float32),
            pltpu.VMEM((ROWS_T,), jnp.float32),
            pltpu.VMEM_SHARED((N_PAD,), jnp.float32),
            pltpu.VMEM_SHARED((N_PAD,), jnp.float32),
            pltpu.SemaphoreType.DMA,
        ],
    )(_hist_body)
    return f(idx4)


def _mm_body(x_ref, w_ref, dega_ref, degb_ref, h_ref, b_ref):
    a = lax.rsqrt(jnp.maximum(dega_ref[0], 1.0))
    h = jnp.dot(x_ref[...], w_ref[...], preferred_element_type=jnp.float32)
    h_ref[...] = h * a
    b_ref[...] = lax.rsqrt(jnp.maximum(degb_ref[...], 1.0))


def _matmul_scale(x_flat, W, dega3, degb2):
    nblk = (2 * N_PAD) // 256
    return pl.pallas_call(
        _mm_body,
        out_shape=(
            jax.ShapeDtypeStruct((2 * N_PAD, H), jnp.float32),
            jax.ShapeDtypeStruct((nblk, 1, 256), jnp.float32),
        ),
        grid=(nblk,),
        in_specs=[
            pl.BlockSpec((256, D), lambda i: (i, 0)),
            pl.BlockSpec((D, H), lambda i: (0, 0)),
            pl.BlockSpec((1, 256, 1), lambda i: (i, 0, 0)),
            pl.BlockSpec((1, 1, 256), lambda i: (i, 0, 0)),
        ],
        out_specs=(
            pl.BlockSpec((256, H), lambda i: (i, 0)),
            pl.BlockSpec((1, 1, 256), lambda i: (i, 0, 0)),
        ),
    )(x_flat, W, dega3, degb2)


def _agg_body(h_hbm, srcsh_hbm, dst_hbm, b_hbm, e_hbm,
              idxs_v, idxd_v, rows_v, red_v, b_v, acc_v, pbuf_v,
              agg_sh, part_sh, gsem0, gsem1, ssem0, ssem1):
    c = lax.axis_index("c")
    s = lax.axis_index("s")

    for r in range(RB):
        for k in range(H // 16):
            red_v[r, pl.ds(k * 16, 16)] = jnp.zeros((16,), jnp.float32)
    for i in range(NRB):
        pltpu.sync_copy(red_v, agg_sh.at[pl.ds(s * ROWS_T + i * RB, RB), :])
    pltpu.sync_copy(red_v.at[0], part_sh.at[s, 0])
    plsc.subcore_barrier()

    gsem = (gsem0, gsem1)
    ssem = (ssem0, ssem1)

    def gather(j, p):
        return pltpu.async_copy(h_hbm.at[idxs_v.at[j]], rows_v.at[p], gsem[p])

    def scatter(j, p):
        return pltpu.async_copy(rows_v.at[p], agg_sh.at[idxd_v.at[j]],
                                ssem[p], add=True)

    def super_body(u, carry):
        base = s * T_TILE + u * SUP
        pltpu.sync_copy(srcsh_hbm.at[c, pl.ds(base, SUP), :], idxs_v)
        pltpu.sync_copy(dst_hbm.at[c, pl.ds(base, SUP), :], idxd_v)
        gather(0, 0)

        def body(q, carry2):
            for p in range(2):
                j = q * 2 + p
                @pl.when(j >= 1)
                def _():
                    pltpu.make_async_copy(
                        rows_v.at[1 - p], agg_sh.at[idxd_v.at[j]], ssem[1 - p]
                    ).wait()

                @pl.when(j + 1 < SUP)
                def _():
                    gather(j + 1, 1 - p)

                pltpu.make_async_copy(
                    h_hbm.at[idxs_v.at[j]], rows_v.at[p], gsem[p]
                ).wait()
                scatter(j, p)
            return carry2

        lax.fori_loop(0, SUP // 2, body, 0)
        pltpu.make_async_copy(
            rows_v.at[1], agg_sh.at[idxd_v.at[SUP - 1]], ssem[1]
        ).wait()
        return carry

    lax.fori_loop(0, T_TILE // SUP, super_body, 0)
    plsc.subcore_barrier()

    pltpu.sync_copy(b_hbm.at[c, 0, pl.ds(s * ROWS_T, ROWS_T)], b_v)

    def red_block(i, acc):
        pltpu.sync_copy(agg_sh.at[pl.ds(s * ROWS_T + i * RB, RB), :], red_v)
        for g in range(RB // 16):
            bv = b_v[pl.ds(i * RB + g * 16, 16)]
            for r in range(16):
                bs = bv[r]
                for k in range(H // 16):
                    v = red_v[g * 16 + r, pl.ds(k * 16, 16)]
                    acc = tuple(
                        acc[q] + jnp.maximum(v, 0.0) * bs if q == k else acc[q]
                        for q in range(H // 16)
                    )
        return acc

    acc0 = tuple(jnp.zeros((16,), jnp.float32) for _ in range(H // 16))
    acc = lax.fori_loop(0, NRB, red_block, acc0)
    for k in range(H // 16):
        acc_v[pl.ds(k * 16, 16)] = acc[k]
    pltpu.sync_copy(acc_v, part_sh.at[s, 0])
    plsc.subcore_barrier()

    @pl.when(s == 0)
    def _():
        pltpu.sync_copy(part_sh, pbuf_v)
        tot = tuple(jnp.zeros((16,), jnp.float32) for _ in range(H // 16))
        for t in range(NT):
            for k in range(H // 16):
                tot = tuple(
                    tot[q] + pbuf_v[t, 0, pl.ds(k * 16, 16)] if q == k else tot[q]
                    for q in range(H // 16)
                )
        for k in range(H // 16):
            acc_v[pl.ds(k * 16, 16)] = tot[k]
        pltpu.sync_copy(acc_v, e_hbm.at[c, 0])


def _aggregate(h_flat, srcsh, dst2, b3):
    mesh = plsc.VectorSubcoreMesh(core_axis_name="c", subcore_axis_name="s")
    f = functools.partial(
        pl.kernel,
        out_type=jax.ShapeDtypeStruct((2, 1, H), jnp.float32),
        mesh=mesh,
        scratch_types=[
            pltpu.VMEM((SUP, CHUNK), jnp.int32),
            pltpu.VMEM((SUP, CHUNK), jnp.int32),
            pltpu.VMEM((2, CHUNK, H), jnp.float32),
            pltpu.VMEM((RB, H), jnp.float32),
            pltpu.VMEM((ROWS_T,), jnp.float32),
            pltpu.VMEM((H,), jnp.float32),
            pltpu.VMEM((NT, 1, H), jnp.float32),
            pltpu.VMEM_SHARED((N_PAD, H), jnp.float32),
            pltpu.VMEM_SHARED((NT, 1, H), jnp.float32),
            pltpu.SemaphoreType.DMA,
            pltpu.SemaphoreType.DMA,
            pltpu.SemaphoreType.DMA,
            pltpu.SemaphoreType.DMA,
        ],
    )(_agg_body)
    return f(h_flat, srcsh, dst2, b3)


def _dot_body(e_ref, o_ref):
    o_ref[...] = jnp.sum(e_ref[0] * e_ref[1], keepdims=True)[None] * (
        1.0 / (float(N) * float(N))
    )


def _dot(e2):
    return pl.pallas_call(
        _dot_body,
        out_shape=jax.ShapeDtypeStruct((1, 1), jnp.float32),
    )(e2)


def _pad_idx(a):
    return jnp.concatenate([a, jnp.full((E_PAD - E,), N, jnp.int32)])


def kernel(x1, x2, edge_index1, edge_index2, W):
    ei1 = edge_index1.astype(jnp.int32)
    ei2 = edge_index2.astype(jnp.int32)
    src1, dst1 = _pad_idx(ei1[0]), _pad_idx(ei1[1])
    src2, dst2 = _pad_idx(ei2[0]), _pad_idx(ei2[1])
    idx4 = jnp.stack([src1, dst1, src2, dst2]).reshape(4, T_PAD, CHUNK)
    srcsh = jnp.stack([src1, src2 + N_PAD]).reshape(2, T_PAD, CHUNK)
    dsts = jnp.stack([dst1, dst2]).reshape(2, T_PAD, CHUNK)

    xp1 = jnp.pad(x1, ((0, N_PAD - N), (0, 0)))
    xp2 = jnp.pad(x2, ((0, N_PAD - N), (0, 0)))
    x_flat = jnp.concatenate([xp1, xp2])

    deg = _degree_hist(idx4).reshape(4, N_PAD)
    dega3 = deg[jnp.array([0, 2])].reshape((2 * N_PAD) // 256, 256, 1)
    degb2 = deg[jnp.array([1, 3])].reshape((2 * N_PAD) // 256, 1, 256)

    h_flat, b2 = _matmul_scale(x_flat, W, dega3, degb2)
    b3 = b2.reshape(2, 1, N_PAD)

    e2 = _aggregate(h_flat, srcsh, dsts, b3)
    return _dot(e2.reshape(2, H))[0, 0]
